# baseline (speedup 1.0000x reference)
import functools

import jax
import jax.numpy as jnp
from jax import lax
from jax.experimental import pallas as pl
from jax.experimental.pallas import tpu as pltpu
from jax.experimental.pallas import tpu_sc as plsc

B, C, H, W = 8, 768, 32, 32
P = H * W
D = 256
K = 8192
TN = 512
NT = K // TN

NC, NS = 2, 16
NW = NC * NS
BPW = (B * P) // NW
CHUNK = 128


def _dist_body(z_ref, w_ref, b_ref, e_ref, et_ref, ind_ref, acc_ref,
               fpm, f2v, best):
    b = pl.program_id(0)
    j = pl.program_id(1)

    @pl.when(jnp.logical_and(b == 0, j == 0))
    def _init_acc():
        acc_ref[0, 0] = 0.0

    @pl.when(j == 0)
    def _project():
        ze = jnp.dot(w_ref[...], z_ref[0],
                     preferred_element_type=jnp.float32) + b_ref[...]
        fv = ze.T
        f2x = 2.0 * fv
        masked = lax.bitcast_convert_type(
            lax.bitcast_convert_type(f2x, jnp.uint32)
            & jnp.uint32(0xFFFF0000), jnp.float32)
        fpm[...] = masked.astype(jnp.bfloat16)
        f2v[...] = jnp.sum(fv * fv, axis=1, keepdims=True)

    et = e_ref[...]
    e2 = jnp.sum(et_ref[...] * et_ref[...], axis=0, keepdims=True)
    c2 = lax.dot_general(fpm[...], et.astype(jnp.bfloat16),
                         (((1,), (1,)), ((), ())),
                         preferred_element_type=jnp.float32)
    s = (f2v[...] - c2) + e2
    m = jnp.min(s, axis=1, keepdims=True)
    cols = lax.broadcasted_iota(jnp.int32, (P, TN), 1) + j * TN
    cand = jnp.min(jnp.where(s == m, cols, jnp.int32(2**30)),
                   axis=1, keepdims=True)

    @pl.when(j == 0)
    def _first():
        best[...] = m
        ind_ref[0] = cand

    @pl.when(j > 0)
    def _rest():
        prev = best[...]
        upd = m < prev
        best[...] = jnp.where(upd, m, prev)
        ind_ref[0] = jnp.where(upd, cand, ind_ref[0])

    @pl.when(j == NT - 1)
    def _finish():
        acc_ref[0, 0] += jnp.sum(best[...])


def _transpose_body(zq_ref, out_ref):
    out_ref[0] = zq_ref[0].T


def _argmin_and_loss(z3, w, b2, e, et):
    return pl.pallas_call(
        _dist_body,
        grid=(B, NT),
        in_specs=[
            pl.BlockSpec((1, C, P), lambda b, j: (b, 0, 0)),
            pl.BlockSpec((D, C), lambda b, j: (0, 0)),
            pl.BlockSpec((D, 1), lambda b, j: (0, 0)),
            pl.BlockSpec((TN, D), lambda b, j: (j, 0)),
            pl.BlockSpec((D, TN), lambda b, j: (0, j)),
        ],
        out_specs=[
            pl.BlockSpec((1, P, 1), lambda b, j: (b, 0, 0)),
            pl.BlockSpec((1, 1), lambda b, j: (0, 0),
                         memory_space=pltpu.SMEM),
        ],
        out_shape=[
            jax.ShapeDtypeStruct((B, P, 1), jnp.int32),
            jax.ShapeDtypeStruct((1, 1), jnp.float32),
        ],
        scratch_shapes=[
            pltpu.VMEM((P, D), jnp.bfloat16),
            pltpu.VMEM((P, 1), jnp.float32),
            pltpu.VMEM((P, 1), jnp.float32),
        ],
    )(z3, w, b2, e, et)


def _sc_gather(table, idx_flat):
    mesh = plsc.VectorSubcoreMesh(core_axis_name="c", subcore_axis_name="s")

    @functools.partial(
        pl.kernel,
        mesh=mesh,
        out_type=jax.ShapeDtypeStruct((B * P, D), jnp.float32),
        scratch_types=[
            pltpu.VMEM((CHUNK,), jnp.int32),
            pltpu.VMEM((CHUNK, D), jnp.float32),
            pltpu.SemaphoreType.DMA,
        ],
    )
    def gather(table_hbm, idx_hbm, out_hbm, idx_v, rows_v, sem):
        wid = lax.axis_index("s") * NC + lax.axis_index("c")
        base = wid * BPW
        for c in range(BPW // CHUNK):
            off = base + c * CHUNK
            pltpu.sync_copy(idx_hbm.at[pl.ds(off, CHUNK)], idx_v)
            pltpu.async_copy(table_hbm.at[idx_v], rows_v, sem).wait()
            pltpu.sync_copy(rows_v, out_hbm.at[pl.ds(off, CHUNK)])

    return gather(table, idx_flat)


def _to_channel_major(zq_pm):
    return pl.pallas_call(
        _transpose_body,
        grid=(B,),
        in_specs=[pl.BlockSpec((1, P, D), lambda b: (b, 0, 0))],
        out_specs=pl.BlockSpec((1, D, P), lambda b: (b, 0, 0)),
        out_shape=jax.ShapeDtypeStruct((B, D, P), jnp.float32),
    )(zq_pm)


def kernel(z, W_proj, b_proj, embed_w):
    z3 = z.reshape(B, C, P)
    b2 = b_proj.reshape(D, 1)

    ind3, acc = _argmin_and_loss(z3, W_proj, b2, embed_w, embed_w.T)
    ind_flat = ind3.reshape(B * P)

    zq_pm = _sc_gather(embed_w, ind_flat)
    zq_cm = _to_channel_major(zq_pm.reshape(B, P, D))

    m = acc[0, 0] / (B * P * D)
    latent_loss = (0.25 * m + m) * 10.0

    z_q = zq_cm.reshape(B, D, H, W)
    ind = ind_flat.reshape(B, H, W)
    return (z_q, latent_loss, ind)
